# baseline (speedup 1.0000x reference)
import functools

import numpy as np
import jax
import jax.numpy as jnp
from jax import lax
from jax.experimental import pallas as pl
from jax.experimental.pallas import tpu as pltpu
from jax.experimental.pallas import tpu_sc as plsc

_NC = 2
_NS = 16
_NW = _NC * _NS


def _corner_table(m):
    nc = (m - 1) ** 4
    tab = np.zeros((nc, 16), np.int32)
    for cell in range(nc):
        f = [(cell // (m - 1) ** d) % (m - 1) for d in range(4)]
        for s in range(16):
            tab[cell, s] = sum((f[d] + ((s >> d) & 1)) * m ** d for d in range(4))
    return tab


def _prep(edge_attr, src, dst, m, Eb, ncells, NB, fill_dst):
    E = edge_attr.shape[0]
    v = edge_attr * (m - 1.0)
    fl = jnp.floor(v)
    fli = jnp.clip(fl.astype(jnp.int32), 0, m - 2)
    frac = v - fl
    strides = jnp.array([(m - 1) ** d for d in range(4)], jnp.int32)
    cell = (fli * strides[None, :]).sum(axis=1)

    R = 256
    Ep = ((E + R - 1) // R) * R
    cell_p = jnp.concatenate(
        [cell, jnp.full((Ep - E,), ncells, jnp.int32)]) if Ep != E else cell
    oh = (cell_p.reshape(Ep // R, R)[:, :, None]
          == jnp.arange(ncells)[None, None, :]).astype(jnp.bfloat16)
    tri = jnp.tril(jnp.ones((R, R), jnp.bfloat16))
    intra = jnp.einsum("rt,ctk->crk", tri, oh,
                       preferred_element_type=jnp.float32)
    chunk_tot = intra[:, -1, :]
    prev = jnp.concatenate(
        [jnp.zeros((1, ncells), jnp.float32), jnp.cumsum(chunk_tot, 0)[:-1]])
    counts = chunk_tot.sum(0)
    pad_counts = jnp.ceil(counts / Eb) * Eb
    pad_offs = jnp.concatenate(
        [jnp.zeros((1,), jnp.float32), jnp.cumsum(pad_counts)[:-1]])
    sel = intra + (prev + pad_offs[None, :])[:, None, :]
    ppos = ((oh.astype(jnp.float32) * sel).sum(-1).reshape(Ep)[:E]
            - 1.0).astype(jnp.int32)

    E_PAD = NB * Eb
    meta = jnp.concatenate(
        [frac, jnp.ones((E, 1), jnp.float32), jnp.zeros((E, 1), jnp.float32),
         lax.bitcast_convert_type(src, jnp.float32)[:, None],
         lax.bitcast_convert_type(dst, jnp.float32)[:, None]], axis=1)
    base = jnp.zeros((8,), jnp.float32).at[7].set(
        lax.bitcast_convert_type(jnp.int32(fill_dst), jnp.float32))
    meta_pad = jnp.broadcast_to(base, (E_PAD, 8)).at[ppos].set(meta)
    src_pad = lax.bitcast_convert_type(meta_pad[:, 6], jnp.int32)
    dst_pad = lax.bitcast_convert_type(meta_pad[:, 7], jnp.int32)
    aux = meta_pad
    blk_start = (jnp.arange(NB, dtype=jnp.float32) * Eb)[:, None]
    blk_cell = jnp.clip(
        (blk_start >= pad_offs[None, :]).astype(jnp.int32).sum(1) - 1,
        0, ncells - 1)
    return src_pad, dst_pad, aux, blk_cell


def _tc_messages(xj, aux, weight, tab2, Eb, D):
    E_PAD = xj.shape[0]
    NB = E_PAD // Eb
    K = weight.shape[0]

    def body(tab_ref, xj_ref, aux_ref, w_ref, out_ref):
        x = xj_ref[...]
        mask = aux_ref[:, 4:5]
        fr = [aux_ref[:, d:d + 1] for d in range(4)]
        om = [1.0 - f for f in fr]
        b = pl.program_id(0)
        acc = jnp.zeros((Eb, D), jnp.float32)
        for s in range(16):
            beta = mask
            for d in range(4):
                beta = beta * (fr[d] if ((s >> d) & 1) else om[d])
            acc = acc + jnp.dot((x * beta).astype(jnp.bfloat16),
                                w_ref[tab_ref[b * 16 + s]],
                                preferred_element_type=jnp.float32)
        out_ref[...] = acc

    grid_spec = pltpu.PrefetchScalarGridSpec(
        num_scalar_prefetch=1,
        grid=(NB,),
        in_specs=[
            pl.BlockSpec((Eb, D), lambda b, tab: (b, 0)),
            pl.BlockSpec((Eb, 8), lambda b, tab: (b, 0)),
            pl.BlockSpec((K, D, D), lambda b, tab: (0, 0, 0)),
        ],
        out_specs=pl.BlockSpec((Eb, D), lambda b, tab: (b, 0)),
    )
    return pl.pallas_call(
        body,
        grid_spec=grid_spec,
        out_shape=jax.ShapeDtypeStruct((E_PAD, D), jnp.float32),
    )(tab2, xj, aux, weight)


def _sc_gather(table, idx, D, CH=128):
    B = idx.shape[0]
    n_tab = table.shape[0]
    assert B % (_NW * CH) == 0 and n_tab % 8 == 0
    b_per_w = B // _NW
    nch = b_per_w // CH
    rpt = (n_tab // _NS) // 8 * 8
    rem = n_tab - rpt * _NS
    mesh = plsc.VectorSubcoreMesh(core_axis_name="c", subcore_axis_name="s")

    @functools.partial(
        pl.kernel, mesh=mesh,
        out_type=jax.ShapeDtypeStruct((B, D), jnp.float32),
        scratch_types=[
            pltpu.VMEM((CH,), jnp.int32),
            pltpu.VMEM((CH, D), jnp.float32),
            pltpu.SemaphoreType.DMA,
            pltpu.VMEM_SHARED((n_tab, D), jnp.float32),
        ],
    )
    def k(table_hbm, idx_hbm, out_hbm, idx_v, rows_v, sem, tbl):
        c = lax.axis_index("c")
        s = lax.axis_index("s")
        pltpu.sync_copy(table_hbm.at[pl.ds(s * rpt, rpt)],
                        tbl.at[pl.ds(s * rpt, rpt)])
        if rem:
            @pl.when(s == 0)
            def _():
                pltpu.sync_copy(table_hbm.at[pl.ds(rpt * _NS, rem)],
                                tbl.at[pl.ds(rpt * _NS, rem)])
        plsc.subcore_barrier()
        base = (c * _NS + s) * b_per_w

        def step(i, carry):
            off = base + i * CH
            pltpu.sync_copy(idx_hbm.at[pl.ds(off, CH)], idx_v)
            pltpu.async_copy(tbl.at[idx_v], rows_v, sem).wait()
            pltpu.sync_copy(rows_v, out_hbm.at[pl.ds(off, CH)])
            return carry

        lax.fori_loop(0, nch, step, 0)

    return k(table, idx)


def _sc_scatter(msgs, dst, zeros, N_ACC, D, CH=128):
    B = msgs.shape[0]
    assert B % (_NW * CH) == 0
    b_per_w = B // _NW
    nch = b_per_w // CH
    mesh = plsc.VectorSubcoreMesh(core_axis_name="c", subcore_axis_name="s")

    @functools.partial(
        pl.kernel, mesh=mesh,
        out_type=jax.ShapeDtypeStruct((_NC, N_ACC, D), jnp.float32),
        scratch_types=[
            pltpu.VMEM((CH,), jnp.int32),
            pltpu.VMEM((CH, D), jnp.float32),
            pltpu.SemaphoreType.DMA,
            pltpu.VMEM_SHARED((N_ACC, D), jnp.float32),
        ],
    )
    def k(msgs_hbm, dst_hbm, zeros_hbm, out_hbm, idx_v, rows_v, sem, acc):
        c = lax.axis_index("c")
        s = lax.axis_index("s")

        @pl.when(s == 0)
        def _():
            pltpu.sync_copy(zeros_hbm, acc)

        plsc.subcore_barrier()

        base = (c * _NS + s) * b_per_w

        def step(i, carry):
            off = base + i * CH
            pltpu.sync_copy(dst_hbm.at[pl.ds(off, CH)], idx_v)
            pltpu.sync_copy(msgs_hbm.at[pl.ds(off, CH)], rows_v)
            pltpu.sync_copy(rows_v, acc.at[idx_v], add=True)
            return carry

        lax.fori_loop(0, nch, step, 0)
        plsc.subcore_barrier()

        @pl.when(s == 0)
        def _():
            pltpu.sync_copy(acc, out_hbm.at[c])

    return k(msgs, dst, zeros)


def _sc_degree(dst, ones, zeros, N_ACC, CH=128):
    B = dst.shape[0]
    assert B % (_NW * CH) == 0
    b_per_w = B // _NW
    nch = b_per_w // CH
    mesh = plsc.VectorSubcoreMesh(core_axis_name="c", subcore_axis_name="s")

    @functools.partial(
        pl.kernel, mesh=mesh,
        out_type=jax.ShapeDtypeStruct((_NC, N_ACC, 128), jnp.float32),
        scratch_types=[
            pltpu.VMEM((CH,), jnp.int32),
            pltpu.VMEM((CH, 128), jnp.float32),
            pltpu.VMEM_SHARED((N_ACC, 128), jnp.float32),
        ],
    )
    def k(dst_hbm, ones_hbm, zeros_hbm, out_hbm, idx_v, ones_v, acc):
        c = lax.axis_index("c")
        s = lax.axis_index("s")
        pltpu.sync_copy(ones_hbm, ones_v)

        @pl.when(s == 0)
        def _():
            pltpu.sync_copy(zeros_hbm, acc)

        plsc.subcore_barrier()

        base = (c * _NS + s) * b_per_w

        def step(i, carry):
            off = base + i * CH
            pltpu.sync_copy(dst_hbm.at[pl.ds(off, CH)], idx_v)
            pltpu.sync_copy(ones_v, acc.at[idx_v], add=True)
            return carry

        lax.fori_loop(0, nch, step, 0)
        plsc.subcore_barrier()

        @pl.when(s == 0)
        def _():
            pltpu.sync_copy(acc, out_hbm.at[c])

    return k(dst, ones, zeros)


def _tc_combine1(P, DEG, x, root, bias, N, D, Nb=1000):
    def body(p_ref, deg_ref, x_ref, r_ref, b_ref, h_ref, di_ref):
        msum = p_ref[0] + p_ref[1]
        deg = deg_ref[0, :, :1] + deg_ref[1, :, :1]
        dinv = 1.0 / jnp.maximum(deg, 1.0)
        h = msum * dinv + jnp.dot(x_ref[...], r_ref[...],
                                  preferred_element_type=jnp.float32) + b_ref[...]
        h_ref[...] = jnp.where(h > 0, h, jnp.exp(h) - 1.0)
        di_ref[...] = jnp.broadcast_to(dinv, (Nb, D))

    return pl.pallas_call(
        body,
        grid=(N // Nb,),
        in_specs=[
            pl.BlockSpec((_NC, Nb, D), lambda b: (0, b, 0)),
            pl.BlockSpec((_NC, Nb, 128), lambda b: (0, b, 0)),
            pl.BlockSpec((Nb, D), lambda b: (b, 0)),
            pl.BlockSpec((D, D), lambda b: (0, 0)),
            pl.BlockSpec((1, D), lambda b: (0, 0)),
        ],
        out_specs=[
            pl.BlockSpec((Nb, D), lambda b: (b, 0)),
            pl.BlockSpec((Nb, D), lambda b: (b, 0)),
        ],
        out_shape=[
            jax.ShapeDtypeStruct((N, D), jnp.float32),
            jax.ShapeDtypeStruct((N, D), jnp.float32),
        ],
    )(P, DEG, x, root, bias.reshape(1, D))


def _tc_combine2(P, h1, deginv, root, bias, m1w, m1b, m2w, m2b, N, D, C, Nb=1000):
    def body(p_ref, h1_ref, di_ref, r_ref, b_ref, w1_ref, b1_ref, w2_ref, b2_ref,
             out_ref):
        msum = p_ref[0] + p_ref[1]
        h = msum * di_ref[...] + jnp.dot(h1_ref[...], r_ref[...],
                                         preferred_element_type=jnp.float32) + b_ref[...]
        h = jnp.where(h > 0, h, jnp.exp(h) - 1.0)
        z = jnp.maximum(jnp.dot(h, w1_ref[...],
                                preferred_element_type=jnp.float32) + b1_ref[...], 0.0)
        out_ref[...] = jnp.maximum(
            jnp.dot(z, w2_ref[...], preferred_element_type=jnp.float32)
            + b2_ref[...], 0.0)

    return pl.pallas_call(
        body,
        grid=(N // Nb,),
        in_specs=[
            pl.BlockSpec((_NC, Nb, D), lambda b: (0, b, 0)),
            pl.BlockSpec((Nb, D), lambda b: (b, 0)),
            pl.BlockSpec((Nb, D), lambda b: (b, 0)),
            pl.BlockSpec((D, D), lambda b: (0, 0)),
            pl.BlockSpec((1, D), lambda b: (0, 0)),
            pl.BlockSpec((D, D), lambda b: (0, 0)),
            pl.BlockSpec((1, D), lambda b: (0, 0)),
            pl.BlockSpec((D, C), lambda b: (0, 0)),
            pl.BlockSpec((1, C), lambda b: (0, 0)),
        ],
        out_specs=pl.BlockSpec((Nb, C), lambda b: (b, 0)),
        out_shape=jax.ShapeDtypeStruct((N, C), jnp.float32),
    )(P, h1, deginv, root, bias.reshape(1, D), m1w, m1b.reshape(1, D),
      m2w, m2b.reshape(1, C))


def _num_blocks(E, Eb, ncells, align):
    nb = (E + Eb - 1) // Eb + ncells
    assert align % Eb == 0
    mult = align // Eb
    return ((nb + mult - 1) // mult) * mult


def kernel(x, edge_index, edge_attr, conv1_weight, conv1_root, conv1_bias,
           conv2_weight, conv2_root, conv2_bias, mlp1_w, mlp1_b, mlp2_w, mlp2_b):
    N, D = x.shape
    E = edge_index.shape[1]
    C = mlp2_w.shape[1]
    src = edge_index[0].astype(jnp.int32)
    dst = edge_index[1].astype(jnp.int32)
    CH = 128
    N_ACC = N + 8
    zeros = jnp.zeros((N_ACC, D), jnp.float32)
    ones = jnp.ones((CH, 128), jnp.float32)

    m1, Eb1 = 3, 512
    nc1 = (m1 - 1) ** 4
    NB1 = _num_blocks(E, Eb1, nc1, _NW * CH)
    src1, dst1, aux1, blk1 = _prep(edge_attr, src, dst, m1, Eb1, nc1, NB1, N)
    tab2_1 = jnp.take(jnp.asarray(_corner_table(m1)), blk1, axis=0).reshape(-1)
    DEG = _sc_degree(dst1, ones, zeros[:, :128], N_ACC, CH)
    xj1 = _sc_gather(x, src1, D, CH)
    msg1 = _tc_messages(xj1, aux1, conv1_weight.astype(jnp.bfloat16),
                        tab2_1, Eb1, D)
    P1 = _sc_scatter(msg1, dst1, zeros, N_ACC, D, CH)
    h1, deginv = _tc_combine1(P1, DEG, x, conv1_root, conv1_bias, N, D)

    m2, Eb2 = 5, 128
    nc2 = (m2 - 1) ** 4
    NB2 = _num_blocks(E, Eb2, nc2, _NW * CH)
    src2, dst2, aux2, blk2 = _prep(edge_attr, src, dst, m2, Eb2, nc2, NB2, N)
    tab2_2 = jnp.take(jnp.asarray(_corner_table(m2)), blk2, axis=0).reshape(-1)
    xj2 = _sc_gather(h1, src2, D, CH)
    msg2 = _tc_messages(xj2, aux2, conv2_weight.astype(jnp.bfloat16),
                        tab2_2, Eb2, D)
    P2 = _sc_scatter(msg2, dst2, zeros, N_ACC, D, CH)
    return _tc_combine2(P2, h1, deginv, conv2_root, conv2_bias,
                        mlp1_w, mlp1_b, mlp2_w, mlp2_b, N, D, C)


# speedup vs baseline: 1.0067x; 1.0067x over previous
import functools

import numpy as np
import jax
import jax.numpy as jnp
from jax import lax
from jax.experimental import pallas as pl
from jax.experimental.pallas import tpu as pltpu
from jax.experimental.pallas import tpu_sc as plsc

_NC = 2
_NS = 16
_NW = _NC * _NS


def _corner_table(m):
    nc = (m - 1) ** 4
    tab = np.zeros((nc, 16), np.int32)
    for cell in range(nc):
        f = [(cell // (m - 1) ** d) % (m - 1) for d in range(4)]
        for s in range(16):
            tab[cell, s] = sum((f[d] + ((s >> d) & 1)) * m ** d for d in range(4))
    return tab


def _prep(edge_attr, src, dst, m, Eb, ncells, NB, fill_dst):
    E = edge_attr.shape[0]
    v = edge_attr * (m - 1.0)
    fl = jnp.floor(v)
    fli = jnp.clip(fl.astype(jnp.int32), 0, m - 2)
    frac = v - fl
    strides = jnp.array([(m - 1) ** d for d in range(4)], jnp.int32)
    cell = (fli * strides[None, :]).sum(axis=1)

    R = 256
    Ep = ((E + R - 1) // R) * R
    cell_p = jnp.concatenate(
        [cell, jnp.full((Ep - E,), ncells, jnp.int32)]) if Ep != E else cell
    oh = (cell_p.reshape(Ep // R, R)[:, :, None]
          == jnp.arange(ncells)[None, None, :]).astype(jnp.bfloat16)
    tri = jnp.tril(jnp.ones((R, R), jnp.bfloat16))
    intra = jnp.einsum("rt,ctk->crk", tri, oh,
                       preferred_element_type=jnp.float32)
    chunk_tot = intra[:, -1, :]
    prev = jnp.concatenate(
        [jnp.zeros((1, ncells), jnp.float32), jnp.cumsum(chunk_tot, 0)[:-1]])
    counts = chunk_tot.sum(0)
    pad_counts = jnp.ceil(counts / Eb) * Eb
    pad_offs = jnp.concatenate(
        [jnp.zeros((1,), jnp.float32), jnp.cumsum(pad_counts)[:-1]])
    sel = intra + (prev + pad_offs[None, :])[:, None, :]
    ppos = ((oh.astype(jnp.float32) * sel).sum(-1).reshape(Ep)[:E]
            - 1.0).astype(jnp.int32)

    E_PAD = NB * Eb
    meta = jnp.concatenate(
        [frac, jnp.ones((E, 1), jnp.float32), jnp.zeros((E, 1), jnp.float32),
         lax.bitcast_convert_type(src, jnp.float32)[:, None],
         lax.bitcast_convert_type(dst, jnp.float32)[:, None]], axis=1)
    base = jnp.zeros((8,), jnp.float32).at[7].set(
        lax.bitcast_convert_type(jnp.int32(fill_dst), jnp.float32))
    meta_pad = jnp.broadcast_to(base, (E_PAD, 8)).at[ppos].set(meta)
    src_pad = lax.bitcast_convert_type(meta_pad[:, 6], jnp.int32)
    dst_pad = lax.bitcast_convert_type(meta_pad[:, 7], jnp.int32)
    aux = meta_pad
    blk_start = (jnp.arange(NB, dtype=jnp.float32) * Eb)[:, None]
    blk_cell = jnp.clip(
        (blk_start >= pad_offs[None, :]).astype(jnp.int32).sum(1) - 1,
        0, ncells - 1)
    return src_pad, dst_pad, aux, blk_cell


def _tc_messages(xj, aux, weight, tab2, Eb, D):
    E_PAD = xj.shape[0]
    NB = E_PAD // Eb
    K = weight.shape[0]

    def body(tab_ref, xj_ref, aux_ref, w_ref, out_ref):
        x = xj_ref[...]
        mask = aux_ref[:, 4:5]
        fr = [aux_ref[:, d:d + 1] for d in range(4)]
        om = [1.0 - f for f in fr]
        b = pl.program_id(0)
        acc = jnp.zeros((Eb, D), jnp.float32)
        for s in range(16):
            beta = mask
            for d in range(4):
                beta = beta * (fr[d] if ((s >> d) & 1) else om[d])
            acc = acc + jnp.dot(x * beta, w_ref[tab_ref[b * 16 + s]],
                                preferred_element_type=jnp.float32)
        out_ref[...] = acc

    grid_spec = pltpu.PrefetchScalarGridSpec(
        num_scalar_prefetch=1,
        grid=(NB,),
        in_specs=[
            pl.BlockSpec((Eb, D), lambda b, tab: (b, 0)),
            pl.BlockSpec((Eb, 8), lambda b, tab: (b, 0)),
            pl.BlockSpec((K, D, D), lambda b, tab: (0, 0, 0)),
        ],
        out_specs=pl.BlockSpec((Eb, D), lambda b, tab: (b, 0)),
    )
    return pl.pallas_call(
        body,
        grid_spec=grid_spec,
        out_shape=jax.ShapeDtypeStruct((E_PAD, D), jnp.float32),
    )(tab2, xj, aux, weight)


def _sc_gather(table, idx, D, CH=128):
    B = idx.shape[0]
    n_tab = table.shape[0]
    assert B % (_NW * CH) == 0 and n_tab % 8 == 0
    b_per_w = B // _NW
    nch = b_per_w // CH
    rpt = (n_tab // _NS) // 8 * 8
    rem = n_tab - rpt * _NS
    mesh = plsc.VectorSubcoreMesh(core_axis_name="c", subcore_axis_name="s")

    @functools.partial(
        pl.kernel, mesh=mesh,
        out_type=jax.ShapeDtypeStruct((B, D), jnp.float32),
        scratch_types=[
            pltpu.VMEM((CH,), jnp.int32),
            pltpu.VMEM((CH, D), jnp.float32),
            pltpu.SemaphoreType.DMA,
            pltpu.VMEM_SHARED((n_tab, D), jnp.float32),
        ],
    )
    def k(table_hbm, idx_hbm, out_hbm, idx_v, rows_v, sem, tbl):
        c = lax.axis_index("c")
        s = lax.axis_index("s")
        pltpu.sync_copy(table_hbm.at[pl.ds(s * rpt, rpt)],
                        tbl.at[pl.ds(s * rpt, rpt)])
        if rem:
            @pl.when(s == 0)
            def _():
                pltpu.sync_copy(table_hbm.at[pl.ds(rpt * _NS, rem)],
                                tbl.at[pl.ds(rpt * _NS, rem)])
        plsc.subcore_barrier()
        base = (c * _NS + s) * b_per_w

        def step(i, carry):
            off = base + i * CH
            pltpu.sync_copy(idx_hbm.at[pl.ds(off, CH)], idx_v)
            pltpu.async_copy(tbl.at[idx_v], rows_v, sem).wait()
            pltpu.sync_copy(rows_v, out_hbm.at[pl.ds(off, CH)])
            return carry

        lax.fori_loop(0, nch, step, 0)

    return k(table, idx)


def _sc_scatter(msgs, dst, zeros, N_ACC, D, CH=128):
    B = msgs.shape[0]
    assert B % (_NW * CH) == 0
    b_per_w = B // _NW
    nch = b_per_w // CH
    mesh = plsc.VectorSubcoreMesh(core_axis_name="c", subcore_axis_name="s")

    @functools.partial(
        pl.kernel, mesh=mesh,
        out_type=jax.ShapeDtypeStruct((_NC, N_ACC, D), jnp.float32),
        scratch_types=[
            pltpu.VMEM((CH,), jnp.int32),
            pltpu.VMEM((CH, D), jnp.float32),
            pltpu.SemaphoreType.DMA,
            pltpu.VMEM_SHARED((N_ACC, D), jnp.float32),
        ],
    )
    def k(msgs_hbm, dst_hbm, zeros_hbm, out_hbm, idx_v, rows_v, sem, acc):
        c = lax.axis_index("c")
        s = lax.axis_index("s")

        @pl.when(s == 0)
        def _():
            pltpu.sync_copy(zeros_hbm, acc)

        plsc.subcore_barrier()

        base = (c * _NS + s) * b_per_w

        def step(i, carry):
            off = base + i * CH
            pltpu.sync_copy(dst_hbm.at[pl.ds(off, CH)], idx_v)
            pltpu.sync_copy(msgs_hbm.at[pl.ds(off, CH)], rows_v)
            pltpu.sync_copy(rows_v, acc.at[idx_v], add=True)
            return carry

        lax.fori_loop(0, nch, step, 0)
        plsc.subcore_barrier()

        @pl.when(s == 0)
        def _():
            pltpu.sync_copy(acc, out_hbm.at[c])

    return k(msgs, dst, zeros)


def _sc_degree(dst, ones, zeros, N_ACC, CH=128):
    B = dst.shape[0]
    assert B % (_NW * CH) == 0
    b_per_w = B // _NW
    nch = b_per_w // CH
    mesh = plsc.VectorSubcoreMesh(core_axis_name="c", subcore_axis_name="s")

    @functools.partial(
        pl.kernel, mesh=mesh,
        out_type=jax.ShapeDtypeStruct((_NC, N_ACC, 128), jnp.float32),
        scratch_types=[
            pltpu.VMEM((CH,), jnp.int32),
            pltpu.VMEM((CH, 128), jnp.float32),
            pltpu.VMEM_SHARED((N_ACC, 128), jnp.float32),
        ],
    )
    def k(dst_hbm, ones_hbm, zeros_hbm, out_hbm, idx_v, ones_v, acc):
        c = lax.axis_index("c")
        s = lax.axis_index("s")
        pltpu.sync_copy(ones_hbm, ones_v)

        @pl.when(s == 0)
        def _():
            pltpu.sync_copy(zeros_hbm, acc)

        plsc.subcore_barrier()

        base = (c * _NS + s) * b_per_w

        def step(i, carry):
            off = base + i * CH
            pltpu.sync_copy(dst_hbm.at[pl.ds(off, CH)], idx_v)
            pltpu.sync_copy(ones_v, acc.at[idx_v], add=True)
            return carry

        lax.fori_loop(0, nch, step, 0)
        plsc.subcore_barrier()

        @pl.when(s == 0)
        def _():
            pltpu.sync_copy(acc, out_hbm.at[c])

    return k(dst, ones, zeros)


def _tc_combine1(P, DEG, x, root, bias, N, D, Nb=1000):
    def body(p_ref, deg_ref, x_ref, r_ref, b_ref, h_ref, di_ref):
        msum = p_ref[0] + p_ref[1]
        deg = deg_ref[0, :, :1] + deg_ref[1, :, :1]
        dinv = 1.0 / jnp.maximum(deg, 1.0)
        h = msum * dinv + jnp.dot(x_ref[...], r_ref[...],
                                  preferred_element_type=jnp.float32) + b_ref[...]
        h_ref[...] = jnp.where(h > 0, h, jnp.exp(h) - 1.0)
        di_ref[...] = jnp.broadcast_to(dinv, (Nb, D))

    return pl.pallas_call(
        body,
        grid=(N // Nb,),
        in_specs=[
            pl.BlockSpec((_NC, Nb, D), lambda b: (0, b, 0)),
            pl.BlockSpec((_NC, Nb, 128), lambda b: (0, b, 0)),
            pl.BlockSpec((Nb, D), lambda b: (b, 0)),
            pl.BlockSpec((D, D), lambda b: (0, 0)),
            pl.BlockSpec((1, D), lambda b: (0, 0)),
        ],
        out_specs=[
            pl.BlockSpec((Nb, D), lambda b: (b, 0)),
            pl.BlockSpec((Nb, D), lambda b: (b, 0)),
        ],
        out_shape=[
            jax.ShapeDtypeStruct((N, D), jnp.float32),
            jax.ShapeDtypeStruct((N, D), jnp.float32),
        ],
    )(P, DEG, x, root, bias.reshape(1, D))


def _tc_combine2(P, h1, deginv, root, bias, m1w, m1b, m2w, m2b, N, D, C, Nb=1000):
    def body(p_ref, h1_ref, di_ref, r_ref, b_ref, w1_ref, b1_ref, w2_ref, b2_ref,
             out_ref):
        msum = p_ref[0] + p_ref[1]
        h = msum * di_ref[...] + jnp.dot(h1_ref[...], r_ref[...],
                                         preferred_element_type=jnp.float32) + b_ref[...]
        h = jnp.where(h > 0, h, jnp.exp(h) - 1.0)
        z = jnp.maximum(jnp.dot(h, w1_ref[...],
                                preferred_element_type=jnp.float32) + b1_ref[...], 0.0)
        out_ref[...] = jnp.maximum(
            jnp.dot(z, w2_ref[...], preferred_element_type=jnp.float32)
            + b2_ref[...], 0.0)

    return pl.pallas_call(
        body,
        grid=(N // Nb,),
        in_specs=[
            pl.BlockSpec((_NC, Nb, D), lambda b: (0, b, 0)),
            pl.BlockSpec((Nb, D), lambda b: (b, 0)),
            pl.BlockSpec((Nb, D), lambda b: (b, 0)),
            pl.BlockSpec((D, D), lambda b: (0, 0)),
            pl.BlockSpec((1, D), lambda b: (0, 0)),
            pl.BlockSpec((D, D), lambda b: (0, 0)),
            pl.BlockSpec((1, D), lambda b: (0, 0)),
            pl.BlockSpec((D, C), lambda b: (0, 0)),
            pl.BlockSpec((1, C), lambda b: (0, 0)),
        ],
        out_specs=pl.BlockSpec((Nb, C), lambda b: (b, 0)),
        out_shape=jax.ShapeDtypeStruct((N, C), jnp.float32),
    )(P, h1, deginv, root, bias.reshape(1, D), m1w, m1b.reshape(1, D),
      m2w, m2b.reshape(1, C))


def _num_blocks(E, Eb, ncells, align):
    nb = (E + Eb - 1) // Eb + ncells
    assert align % Eb == 0
    mult = align // Eb
    return ((nb + mult - 1) // mult) * mult


def kernel(x, edge_index, edge_attr, conv1_weight, conv1_root, conv1_bias,
           conv2_weight, conv2_root, conv2_bias, mlp1_w, mlp1_b, mlp2_w, mlp2_b):
    N, D = x.shape
    E = edge_index.shape[1]
    C = mlp2_w.shape[1]
    src = edge_index[0].astype(jnp.int32)
    dst = edge_index[1].astype(jnp.int32)
    CH = 128
    N_ACC = N + 8
    zeros = jnp.zeros((N_ACC, D), jnp.float32)
    ones = jnp.ones((CH, 128), jnp.float32)

    m1, Eb1 = 3, 512
    nc1 = (m1 - 1) ** 4
    NB1 = _num_blocks(E, Eb1, nc1, _NW * CH)
    src1, dst1, aux1, blk1 = _prep(edge_attr, src, dst, m1, Eb1, nc1, NB1, N)
    tab2_1 = jnp.take(jnp.asarray(_corner_table(m1)), blk1, axis=0).reshape(-1)
    DEG = _sc_degree(dst1, ones, zeros[:, :128], N_ACC, CH)
    xj1 = _sc_gather(x, src1, D, CH)
    msg1 = _tc_messages(xj1, aux1, conv1_weight, tab2_1, Eb1, D)
    P1 = _sc_scatter(msg1, dst1, zeros, N_ACC, D, CH)
    h1, deginv = _tc_combine1(P1, DEG, x, conv1_root, conv1_bias, N, D)

    m2, Eb2 = 5, 128
    nc2 = (m2 - 1) ** 4
    NB2 = _num_blocks(E, Eb2, nc2, _NW * CH)
    src2, dst2, aux2, blk2 = _prep(edge_attr, src, dst, m2, Eb2, nc2, NB2, N)
    tab2_2 = jnp.take(jnp.asarray(_corner_table(m2)), blk2, axis=0).reshape(-1)
    xj2 = _sc_gather(h1, src2, D, CH)
    msg2 = _tc_messages(xj2, aux2, conv2_weight, tab2_2, Eb2, D)
    P2 = _sc_scatter(msg2, dst2, zeros, N_ACC, D, CH)
    return _tc_combine2(P2, h1, deginv, conv2_root, conv2_bias,
                        mlp1_w, mlp1_b, mlp2_w, mlp2_b, N, D, C)
